# baseline (speedup 1.0000x reference)
import dataclasses
import functools

import jax
import jax.numpy as jnp
from jax import lax
from jax.experimental import pallas as pl
from jax.experimental.pallas import tpu as pltpu
from jax.experimental.pallas import tpu_sc as plsc

NC = 2
NS = 16
NW = NC * NS
L = 16


def _sc_bag_sums(text1d, offs_pad, table2, zeros_init, *, N, B, D, nch, ch, S):
    tpw = nch * ch
    arows = B
    rps = arows // NS
    mesh = plsc.VectorSubcoreMesh(core_axis_name="c", subcore_axis_name="s")
    nbits = B.bit_length() - 1
    cp = pltpu.CompilerParams()
    if "needs_layout_passes" in pltpu.CompilerParams.__dataclass_fields__:
        cp = dataclasses.replace(cp, needs_layout_passes=False)
    if "use_tc_tiling_on_sc" in pltpu.CompilerParams.__dataclass_fields__:
        cp = dataclasses.replace(cp, use_tc_tiling_on_sc=False)

    GRP = 5
    ngrp = nch // GRP
    assert nch == ngrp * GRP and ngrp % 2 == 0
    NSLOT = 2 * GRP

    @functools.partial(
        pl.kernel,
        mesh=mesh,
        compiler_params=cp,
        out_type=jax.ShapeDtypeStruct((NC, arows, 1, D), jnp.float32),
        scratch_types=[
            pltpu.VMEM((nch * ch,), jnp.int32),
            pltpu.VMEM((NSLOT, ch), jnp.int32),
            pltpu.VMEM((NSLOT, ch), jnp.int32),
            pltpu.VMEM((NSLOT, ch, 1, D), jnp.float32),
            pltpu.VMEM((2 * B,), jnp.int32),
            pltpu.VMEM_SHARED((arows, 1, D), jnp.float32),
            pltpu.SemaphoreType.DMA((NSLOT,)),
            pltpu.SemaphoreType.DMA((NSLOT,)),
        ],
    )
    def sc_kernel(text_hbm, offs_hbm, table_hbm, zeros_hbm, out_hbm,
                  idx_v, gidx_v, seg_v, rows_v, offs_v, acc_sh, g_sem, s_sem):
        cid = lax.axis_index("c")
        sid = lax.axis_index("s")
        wid = sid * NC + cid
        pltpu.sync_copy(text_hbm.at[pl.ds(wid * tpw, tpw)], idx_v)
        pltpu.sync_copy(offs_hbm, offs_v)
        pltpu.sync_copy(zeros_hbm.at[pl.ds(sid * rps, rps)],
                        acc_sh.at[pl.ds(sid * rps, rps)])
        plsc.subcore_barrier()

        base = wid * tpw
        iota = lax.iota(jnp.int32, L)


        def issue_gathers(grp, bank):
            for j in range(GRP):
                slot = bank * GRP + j
                c = grp * GRP + j
                for g in range(ch // L):
                    t = idx_v[pl.ds(c * ch + g * L, L)]
                    gidx_v[slot, pl.ds(g * L, L)] = jnp.where(
                        t < S, 2 * t, 2 * (t - S) + 1)
                pltpu.async_copy(table_hbm.at[gidx_v.at[slot]],
                                 rows_v.at[slot], g_sem.at[slot])

        def drain_scatters(bank):
            for j in range(GRP):
                slot = bank * GRP + j
                pltpu.make_async_copy(rows_v.at[slot],
                                      acc_sh.at[seg_v.at[slot]],
                                      s_sem.at[slot]).wait()

        def compute_seg(c, slot):
            ps = [base + c * ch + g * L + iota for g in range(ch // L)]
            rs = [jnp.zeros((L,), jnp.int32) for _ in range(ch // L)]
            for k in range(nbits, -1, -1):
                bit = 1 << k
                for g in range(ch // L):
                    cand = rs[g] + bit
                    v = plsc.load_gather(offs_v, [cand - 1])
                    rs[g] = jnp.where(v <= ps[g], cand, rs[g])
            for g in range(ch // L):
                seg_v[slot, pl.ds(g * L, L)] = rs[g] - 1

        issue_gathers(0, 0)

        @pl.loop(0, ngrp, step=2)
        def _(ki):
            for bank in (0, 1):
                k = ki + bank
                ob = 1 - bank

                @pl.when(k >= 1)
                def _():
                    drain_scatters(ob)

                @pl.when(k + 1 < ngrp)
                def _():
                    issue_gathers(k + 1, ob)

                for j in range(GRP):
                    slot = bank * GRP + j
                    c = k * GRP + j
                    pltpu.make_async_copy(
                        table_hbm.at[gidx_v.at[slot]],
                        rows_v.at[slot], g_sem.at[slot]).wait()
                    compute_seg(c, slot)
                    pltpu.async_copy(rows_v.at[slot],
                                     acc_sh.at[seg_v.at[slot]],
                                     s_sem.at[slot], add=True)

        drain_scatters(1)
        plsc.subcore_barrier()
        pltpu.sync_copy(acc_sh.at[pl.ds(sid * rps, rps)],
                        out_hbm.at[cid, pl.ds(sid * rps, rps)])

    return sc_kernel(text1d, offs_pad, table2, zeros_init)


def _tc_repack(tableT, *, V, D):
    S = 500224
    CW = 512
    grid = S // CW

    def body(in_lo, in_hi, out_ref):
        out_ref[:, :D] = lax.transpose(in_lo[...], (1, 0))
        out_ref[:, D:] = lax.transpose(in_hi[...], (1, 0))

    return pl.pallas_call(
        body,
        grid=(grid,),
        in_specs=[pl.BlockSpec((D, CW), lambda i: (0, i)),
                  pl.BlockSpec((D, CW), lambda i: (0, i + grid))],
        out_specs=pl.BlockSpec((CW, 2 * D), lambda i: (i, 0)),
        out_shape=jax.ShapeDtypeStruct((S, 2 * D), jnp.float32),
    )(tableT, tableT), S


def _tc_head(acc2, counts, W, b2, *, B, D, C):
    def body(acc_ref, cnt_ref, w_ref, b_ref, out_ref):
        sums = acc_ref[0, :, 0, :] + acc_ref[1, :, 0, :]
        inv = 1.0 / jnp.maximum(cnt_ref[...], 1.0)
        mean = sums * inv
        out_ref[...] = lax.dot_general(
            mean, w_ref[...], (((1,), (1,)), ((), ())),
            preferred_element_type=jnp.float32) + b_ref[...]

    return pl.pallas_call(
        body,
        out_shape=jax.ShapeDtypeStruct((B, C), jnp.float32),
    )(acc2, counts, W, b2)


def kernel(text, offsets, emb_table, W, b):
    N = text.shape[0]
    B = offsets.shape[0]
    D = emb_table.shape[1]
    C = W.shape[0]
    ch = 64
    nch = N // (NW * ch)
    assert N == NW * nch * ch

    table2, S = _tc_repack(emb_table.T, V=emb_table.shape[0], D=D)
    table2 = table2.reshape(2 * S, 1, D)
    offs_pad = jnp.concatenate(
        [offsets, jnp.full((B,), N, jnp.int32)]).astype(jnp.int32)
    counts = jnp.diff(
        jnp.concatenate([offsets, jnp.array([N], jnp.int32)])
    ).astype(jnp.float32).reshape(B, 1)
    zeros_init = jnp.zeros((B, 1, D), jnp.float32)

    acc2 = _sc_bag_sums(text, offs_pad, table2, zeros_init,
                        N=N, B=B, D=D, nch=nch, ch=ch, S=S)
    return _tc_head(acc2, counts, W, b.reshape(1, C), B=B, D=D, C=C)


# speedup vs baseline: 7.7722x; 7.7722x over previous
import dataclasses
import functools

import jax
import jax.numpy as jnp
from jax import lax
from jax.experimental import pallas as pl
from jax.experimental.pallas import tpu as pltpu
from jax.experimental.pallas import tpu_sc as plsc

NC = 2
NS = 16
NW = NC * NS
L = 16


def _sc_bag_sums(text1d, offs_pad, table2, zeros_init, *, N, B, D, nch, ch):
    tpw = nch * ch
    arows = B
    rps = arows // NS
    mesh = plsc.VectorSubcoreMesh(core_axis_name="c", subcore_axis_name="s")
    nbits = B.bit_length() - 1
    cp = pltpu.CompilerParams()
    if "needs_layout_passes" in pltpu.CompilerParams.__dataclass_fields__:
        cp = dataclasses.replace(cp, needs_layout_passes=False)
    if "use_tc_tiling_on_sc" in pltpu.CompilerParams.__dataclass_fields__:
        cp = dataclasses.replace(cp, use_tc_tiling_on_sc=False)

    GRP = 5
    ngrp = nch // GRP
    assert nch == ngrp * GRP and ngrp % 2 == 0
    NSLOT = 2 * GRP

    @functools.partial(
        pl.kernel,
        mesh=mesh,
        compiler_params=cp,
        out_type=jax.ShapeDtypeStruct((NC, arows, 2 * D), jnp.float32),
        scratch_types=[
            pltpu.VMEM((nch * ch,), jnp.int32),
            pltpu.VMEM((NSLOT, ch), jnp.int32),
            pltpu.VMEM((NSLOT, ch), jnp.int32),
            pltpu.VMEM((NSLOT, ch, 2 * D), jnp.float32),
            pltpu.VMEM((2 * B,), jnp.int32),
            pltpu.VMEM_SHARED((arows, 2 * D), jnp.float32),
            pltpu.SemaphoreType.DMA((NSLOT,)),
            pltpu.SemaphoreType.DMA((NSLOT,)),
        ],
    )
    def sc_kernel(text_hbm, offs_hbm, table_hbm, zeros_hbm, out_hbm,
                  idx_v, gidx_v, seg_v, rows_v, offs_v, acc_sh, g_sem, s_sem):
        cid = lax.axis_index("c")
        sid = lax.axis_index("s")
        wid = sid * NC + cid
        pltpu.sync_copy(text_hbm.at[pl.ds(wid * tpw, tpw)], idx_v)
        pltpu.sync_copy(offs_hbm, offs_v)
        pltpu.sync_copy(zeros_hbm.at[pl.ds(sid * rps, rps)],
                        acc_sh.at[pl.ds(sid * rps, rps)])
        plsc.subcore_barrier()

        base = wid * tpw
        iota = lax.iota(jnp.int32, L)


        def issue_gathers(grp, bank):
            for j in range(GRP):
                slot = bank * GRP + j
                c = grp * GRP + j
                for g in range(ch // L):
                    gidx_v[slot, pl.ds(g * L, L)] = idx_v[
                        pl.ds(c * ch + g * L, L)]
                pltpu.async_copy(table_hbm.at[gidx_v.at[slot]],
                                 rows_v.at[slot], g_sem.at[slot])

        def drain_scatters(bank):
            for j in range(GRP):
                slot = bank * GRP + j
                pltpu.make_async_copy(rows_v.at[slot],
                                      acc_sh.at[seg_v.at[slot]],
                                      s_sem.at[slot]).wait()

        def compute_seg(c, slot):
            ps = [base + c * ch + g * L + iota for g in range(ch // L)]
            rs = [jnp.zeros((L,), jnp.int32) for _ in range(ch // L)]
            for k in range(nbits, -1, -1):
                bit = 1 << k
                for g in range(ch // L):
                    cand = rs[g] + bit
                    v = plsc.load_gather(offs_v, [cand - 1])
                    rs[g] = jnp.where(v <= ps[g], cand, rs[g])
            for g in range(ch // L):
                seg_v[slot, pl.ds(g * L, L)] = rs[g] - 1

        issue_gathers(0, 0)

        @pl.loop(0, ngrp, step=2)
        def _(ki):
            for bank in (0, 1):
                k = ki + bank
                ob = 1 - bank

                @pl.when(k >= 1)
                def _():
                    drain_scatters(ob)

                @pl.when(k + 1 < ngrp)
                def _():
                    issue_gathers(k + 1, ob)

                for j in range(GRP):
                    slot = bank * GRP + j
                    c = k * GRP + j
                    pltpu.make_async_copy(
                        table_hbm.at[gidx_v.at[slot]],
                        rows_v.at[slot], g_sem.at[slot]).wait()
                    compute_seg(c, slot)
                    pltpu.async_copy(rows_v.at[slot],
                                     acc_sh.at[seg_v.at[slot]],
                                     s_sem.at[slot], add=True)

        drain_scatters(1)
        plsc.subcore_barrier()
        pltpu.sync_copy(acc_sh.at[pl.ds(sid * rps, rps)],
                        out_hbm.at[cid, pl.ds(sid * rps, rps)])

    return sc_kernel(text1d, offs_pad, table2, zeros_init)


def _tc_repack(tableT, *, V, D):
    CW = 8192
    grid = (V + CW - 1) // CW

    def body(in_ref, eye_ref, out_ref):
        out_ref[:, :D] = lax.dot_general(
            in_ref[...], eye_ref[...], (((0,), (0,)), ((), ())),
            preferred_element_type=jnp.float32)

    return pl.pallas_call(
        body,
        grid=(grid,),
        in_specs=[pl.BlockSpec((D, CW), lambda i: (0, i)),
                  pl.BlockSpec((D, D), lambda i: (0, 0))],
        out_specs=pl.BlockSpec((CW, 2 * D), lambda i: (i, 0)),
        out_shape=jax.ShapeDtypeStruct((V, 2 * D), jnp.float32),
    )(tableT, jnp.eye(D, dtype=jnp.float32))


def _tc_head(acc2, counts, W, b2, *, B, D, C):
    def body(acc_ref, cnt_ref, w_ref, b_ref, out_ref):
        sums = acc_ref[0, :, :D] + acc_ref[1, :, :D]
        inv = 1.0 / jnp.maximum(cnt_ref[...], 1.0)
        mean = sums * inv
        out_ref[...] = lax.dot_general(
            mean, w_ref[...], (((1,), (1,)), ((), ())),
            preferred_element_type=jnp.float32) + b_ref[...]

    return pl.pallas_call(
        body,
        out_shape=jax.ShapeDtypeStruct((B, C), jnp.float32),
    )(acc2, counts, W, b2)


def kernel(text, offsets, emb_table, W, b):
    N = text.shape[0]
    B = offsets.shape[0]
    D = emb_table.shape[1]
    C = W.shape[0]
    ch = 64
    nch = N // (NW * ch)
    assert N == NW * nch * ch

    table2 = _tc_repack(emb_table.T, V=emb_table.shape[0], D=D)
    offs_pad = jnp.concatenate(
        [offsets, jnp.full((B,), N, jnp.int32)]).astype(jnp.int32)
    counts = jnp.diff(
        jnp.concatenate([offsets, jnp.array([N], jnp.int32)])
    ).astype(jnp.float32).reshape(B, 1)
    zeros_init = jnp.zeros((B, 2 * D), jnp.float32)

    acc2 = _sc_bag_sums(text, offs_pad, table2, zeros_init,
                        N=N, B=B, D=D, nch=nch, ch=ch)
    return _tc_head(acc2, counts, W, b.reshape(1, C), B=B, D=D, C=C)
